# baseline (speedup 1.0000x reference)
import functools

import jax
import jax.numpy as jnp
from jax import lax
from jax.experimental import pallas as pl
from jax.experimental.pallas import tpu as pltpu
from jax.experimental.pallas import tpu_sc as plsc

_B = 16384
_D = 20
_NC = 2
_NS = 16
_NW = _NC * _NS
_BPW = _B // _NW
_L = 16
_NG = _BPW // _L


@functools.partial(
    pl.kernel,
    mesh=plsc.VectorSubcoreMesh(core_axis_name="c", subcore_axis_name="s"),
    out_type=jax.ShapeDtypeStruct((_D, _B), jnp.float32),
    scratch_types=[
        pltpu.VMEM((_BPW,), jnp.int32),
        pltpu.VMEM((_BPW,), jnp.int32),
        pltpu.VMEM((192,), jnp.float32),
        pltpu.VMEM((_D, _BPW), jnp.float32),
        pltpu.SemaphoreType.DMA,
    ],
    compiler_params=pltpu.CompilerParams(
        use_tc_tiling_on_sc=False,
        needs_layout_passes=False,
    ),
)
def _distance_sc(lengths_hbm, table_hbm, out_hbm, len_v, idx_v, tab_v, out_v, sem):
    wid = lax.axis_index("s") * _NC + lax.axis_index("c")
    base = wid * _BPW
    cp_len = pltpu.async_copy(lengths_hbm.at[pl.ds(base, _BPW)], len_v, sem)
    cp_tab = pltpu.async_copy(table_hbm, tab_v.at[pl.ds(0, _D * 9)], sem)
    cp_len.wait()
    cp_tab.wait()
    zero = jnp.full((_L,), 0, jnp.int32)
    one = jnp.full((_L,), 1, jnp.int32)
    four = jnp.full((_L,), 4, jnp.int32)
    twenty = jnp.full((_L,), _D, jnp.int32)
    cap = jnp.full((_L,), 127, jnp.int32)
    lane = lax.iota(jnp.int32, _L)
    for i in range(8):
        lv = lane + jnp.full((_L,), i * _L, jnp.int32)
        acc = jnp.minimum(jnp.maximum(lv, zero), four)
        for k in (3, 4, 5, 6):
            kv = jnp.full((_L,), k, jnp.int32)
            sh = lax.shift_right_arithmetic(lv, kv)
            acc = acc + jnp.minimum(jnp.maximum(sh, zero), one)
        idx_v[pl.ds(i * _L, _L)] = acc * twenty

    dsplat = [jnp.full((_L,), d, jnp.int32) for d in range(_D)]

    def gather_group(g):
        lv = len_v[pl.ds(g * _L, _L)]
        rowb = plsc.load_gather(
            idx_v, [jnp.minimum(jnp.maximum(lv, zero), cap)]
        )
        for d in range(_D):
            val = plsc.load_gather(tab_v, [rowb + dsplat[d]])
            out_v[d, pl.ds(g * _L, _L)] = val

    plsc.parallel_loop(0, _NG, unroll=2)(gather_group)
    cp_out = pltpu.async_copy(out_v, out_hbm.at[:, pl.ds(base, _BPW)], sem)
    cp_out.wait()


def kernel(lengths, table):
    out_t = _distance_sc(lengths, table.reshape(-1))
    return out_t.T


# speedup vs baseline: 1.0044x; 1.0044x over previous
import functools

import jax
import jax.numpy as jnp
from jax import lax
from jax.experimental import pallas as pl
from jax.experimental.pallas import tpu as pltpu
from jax.experimental.pallas import tpu_sc as plsc

_B = 16384
_D = 20
_NC = 2
_NS = 16
_NW = _NC * _NS
_BPW = _B // _NW
_L = 16
_NG = _BPW // _L


@functools.partial(
    pl.kernel,
    mesh=plsc.VectorSubcoreMesh(core_axis_name="c", subcore_axis_name="s"),
    out_type=jax.ShapeDtypeStruct((_D, _B), jnp.float32),
    scratch_types=[
        pltpu.VMEM((_BPW,), jnp.int32),
        pltpu.VMEM((_BPW,), jnp.int32),
        pltpu.VMEM((192,), jnp.float32),
        pltpu.VMEM((_D, _BPW), jnp.float32),
        pltpu.SemaphoreType.DMA,
    ],
    compiler_params=pltpu.CompilerParams(
        use_tc_tiling_on_sc=False,
        needs_layout_passes=False,
    ),
)
def _distance_sc(lengths_hbm, table_hbm, out_hbm, len_v, idx_v, tab_v, out_v, sem):
    wid = lax.axis_index("s") * _NC + lax.axis_index("c")
    base = wid * _BPW
    cp_len = pltpu.async_copy(lengths_hbm.at[pl.ds(base, _BPW)], len_v, sem)
    cp_tab = pltpu.async_copy(table_hbm, tab_v.at[pl.ds(0, _D * 9)], sem)
    zero = jnp.full((_L,), 0, jnp.int32)
    one = jnp.full((_L,), 1, jnp.int32)
    four = jnp.full((_L,), 4, jnp.int32)
    twenty = jnp.full((_L,), _D, jnp.int32)
    cap = jnp.full((_L,), 127, jnp.int32)
    lane = lax.iota(jnp.int32, _L)
    for i in range(8):
        lv = lane + jnp.full((_L,), i * _L, jnp.int32)
        acc = jnp.minimum(jnp.maximum(lv, zero), four)
        for k in (3, 4, 5, 6):
            kv = jnp.full((_L,), k, jnp.int32)
            sh = lax.shift_right_arithmetic(lv, kv)
            acc = acc + jnp.minimum(jnp.maximum(sh, zero), one)
        idx_v[pl.ds(i * _L, _L)] = acc * twenty

    dsplat = [jnp.full((_L,), d, jnp.int32) for d in range(_D)]
    cp_len.wait()
    cp_tab.wait()

    def gather_group(g):
        lv = len_v[pl.ds(g * _L, _L)]
        rowb = plsc.load_gather(
            idx_v, [jnp.minimum(jnp.maximum(lv, zero), cap)]
        )
        for d in range(_D):
            val = plsc.load_gather(tab_v, [rowb + dsplat[d]])
            out_v[d, pl.ds(g * _L, _L)] = val

    plsc.parallel_loop(0, _NG, unroll=2)(gather_group)
    cp_out = pltpu.async_copy(out_v, out_hbm.at[:, pl.ds(base, _BPW)], sem)
    cp_out.wait()


def kernel(lengths, table):
    out_t = _distance_sc(lengths, table.reshape(-1))
    return out_t.T
